# Initial kernel scaffold; baseline (speedup 1.0000x reference)
import jax
import jax.numpy as jnp
from jax.experimental import pallas as pl


def kernel(embeddings, labels, prototypes, prototype_labels):
    raise NotImplementedError("write your pallas kernel here")


# speedup vs baseline: 3.4315x; 3.4315x over previous
import functools

import jax
import jax.numpy as jnp
from jax.experimental import pallas as pl

MARGIN = 0.2
EPS = 1e-07

_B = 4096
_N = 5096
_NP = 5120
_BM = 512


def _triplet_kernel(emb_ref, all_ref, lab_ref, all_lab_ref, sum_ref, cnt_ref):
    i = pl.program_id(0)

    sim = jax.lax.dot_general(
        emb_ref[...], all_ref[...],
        dimension_numbers=(((1,), (1,)), ((), ())),
        preferred_element_type=jnp.float32,
    )
    dist = 1.0 - jnp.clip(sim, -1.0 + EPS, 1.0 - EPS)

    row = i * _BM + jax.lax.broadcasted_iota(jnp.int32, (_BM, _NP), 0)
    col = jax.lax.broadcasted_iota(jnp.int32, (_BM, _NP), 1)
    in_range = col < _N

    lab_eq = lab_ref[...] == all_lab_ref[...]
    pos_mask = lab_eq & (col != row) & in_range
    neg_mask = (~lab_eq) & in_range

    neg_inf = jnp.float32(-jnp.inf)
    pos_inf = jnp.float32(jnp.inf)
    d_ap = jnp.max(jnp.where(pos_mask, dist, neg_inf), axis=1)
    d_an = jnp.min(jnp.where(neg_mask, dist, pos_inf), axis=1)

    valid = jnp.any(pos_mask, axis=1) & jnp.any(neg_mask, axis=1)
    per = jnp.maximum(d_ap - d_an + MARGIN, 0.0)
    per = jnp.where(valid, per, 0.0)

    @pl.when(i == 0)
    def _init():
        sum_ref[...] = jnp.zeros_like(sum_ref)
        cnt_ref[...] = jnp.zeros_like(cnt_ref)

    sum_ref[...] = sum_ref[...] + jnp.sum(per).reshape(1, 1)
    cnt_ref[...] = cnt_ref[...] + jnp.sum(valid.astype(jnp.float32)).reshape(1, 1)


@jax.jit
def kernel(embeddings, labels, prototypes, prototype_labels):
    all_emb = jnp.concatenate(
        [embeddings, prototypes,
         jnp.zeros((_NP - _N, embeddings.shape[1]), jnp.float32)], axis=0)
    all_lab = jnp.concatenate(
        [labels, prototype_labels,
         jnp.zeros((_NP - _N,), labels.dtype)], axis=0)
    lab2d = labels.astype(jnp.int32).reshape(_B, 1)
    all_lab2d = all_lab.astype(jnp.int32).reshape(1, _NP)

    grid = _B // _BM
    s, c = pl.pallas_call(
        _triplet_kernel,
        grid=(grid,),
        in_specs=[
            pl.BlockSpec((_BM, 64), lambda i: (i, 0)),
            pl.BlockSpec((_NP, 64), lambda i: (0, 0)),
            pl.BlockSpec((_BM, 1), lambda i: (i, 0)),
            pl.BlockSpec((1, _NP), lambda i: (0, 0)),
        ],
        out_specs=[
            pl.BlockSpec((1, 1), lambda i: (0, 0)),
            pl.BlockSpec((1, 1), lambda i: (0, 0)),
        ],
        out_shape=[
            jax.ShapeDtypeStruct((1, 1), jnp.float32),
            jax.ShapeDtypeStruct((1, 1), jnp.float32),
        ],
    )(embeddings, all_emb, lab2d, all_lab2d)

    return (s[0, 0] / jnp.maximum(c[0, 0], 1.0)).astype(jnp.float32)


# speedup vs baseline: 4.5652x; 1.3304x over previous
import functools

import jax
import jax.numpy as jnp
from jax.experimental import pallas as pl

MARGIN = 0.2
EPS = 1e-07

_B = 4096
_N = 5096
_NP = 5120
_BM = 512


def _triplet_kernel(emb_ref, all_ref, lab_ref, all_lab_ref, sum_ref, cnt_ref):
    i = pl.program_id(0)

    sim = jax.lax.dot_general(
        emb_ref[...], all_ref[...],
        dimension_numbers=(((1,), (1,)), ((), ())),
        preferred_element_type=jnp.float32,
    )
    row = i * _BM + jax.lax.broadcasted_iota(jnp.int32, (_BM, _NP), 0)
    col = jax.lax.broadcasted_iota(jnp.int32, (_BM, _NP), 1)
    in_range = col < _N

    lab_eq = lab_ref[...] == all_lab_ref[...]
    pos_mask = lab_eq & (col != row) & in_range
    neg_mask = (~lab_eq) & in_range

    min_pos = jnp.min(jnp.where(pos_mask, sim, 3.0), axis=1)
    max_neg = jnp.max(jnp.where(neg_mask, sim, -3.0), axis=1)

    d_ap = 1.0 - jnp.clip(min_pos, -1.0 + EPS, 1.0 - EPS)
    d_an = 1.0 - jnp.clip(max_neg, -1.0 + EPS, 1.0 - EPS)
    valid = (min_pos < 2.0) & (max_neg > -2.0)
    per = jnp.where(valid, jnp.maximum(d_ap - d_an + MARGIN, 0.0), 0.0)

    @pl.when(i == 0)
    def _init():
        sum_ref[...] = jnp.zeros_like(sum_ref)
        cnt_ref[...] = jnp.zeros_like(cnt_ref)

    sum_ref[...] = sum_ref[...] + jnp.sum(per).reshape(1, 1)
    cnt_ref[...] = cnt_ref[...] + jnp.sum(valid.astype(jnp.float32)).reshape(1, 1)


@jax.jit
def kernel(embeddings, labels, prototypes, prototype_labels):
    all_emb = jnp.concatenate(
        [embeddings, prototypes,
         jnp.zeros((_NP - _N, embeddings.shape[1]), jnp.float32)], axis=0)
    all_lab = jnp.concatenate(
        [labels, prototype_labels,
         jnp.zeros((_NP - _N,), labels.dtype)], axis=0)
    lab2d = labels.astype(jnp.int32).reshape(_B, 1)
    all_lab2d = all_lab.astype(jnp.int32).reshape(1, _NP)

    grid = _B // _BM
    s, c = pl.pallas_call(
        _triplet_kernel,
        grid=(grid,),
        in_specs=[
            pl.BlockSpec((_BM, 64), lambda i: (i, 0)),
            pl.BlockSpec((_NP, 64), lambda i: (0, 0)),
            pl.BlockSpec((_BM, 1), lambda i: (i, 0)),
            pl.BlockSpec((1, _NP), lambda i: (0, 0)),
        ],
        out_specs=[
            pl.BlockSpec((1, 1), lambda i: (0, 0)),
            pl.BlockSpec((1, 1), lambda i: (0, 0)),
        ],
        out_shape=[
            jax.ShapeDtypeStruct((1, 1), jnp.float32),
            jax.ShapeDtypeStruct((1, 1), jnp.float32),
        ],
    )(embeddings, all_emb, lab2d, all_lab2d)

    return (s[0, 0] / jnp.maximum(c[0, 0], 1.0)).astype(jnp.float32)
